# baseline (speedup 1.0000x reference)
import functools

import jax
import jax.numpy as jnp
from jax import lax
from jax.experimental import pallas as pl
from jax.experimental.pallas import tpu as pltpu
from jax.experimental.pallas import tpu_sc as plsc

BATCH = 16384
EMBED = 32
CHUNK = 128


def _make_gather(num_users, num_movies):
  info = plsc.get_sparse_core_info()
  nc, ns = info.num_cores, info.num_subcores
  nw = nc * ns
  b_per_w = BATCH // nw
  n_chunks = b_per_w // CHUNK
  rows_per_chunk = BATCH // CHUNK

  mesh = plsc.VectorSubcoreMesh(core_axis_name="c", subcore_axis_name="s")

  @functools.partial(
      pl.kernel,
      mesh=mesh,
      compiler_params=pltpu.CompilerParams(use_tc_tiling_on_sc=False),
      out_type=[
          jax.ShapeDtypeStruct((BATCH, EMBED), jnp.float32),
          jax.ShapeDtypeStruct((BATCH, EMBED), jnp.float32),
      ],
      scratch_types=[
          pltpu.VMEM((n_chunks, CHUNK), jnp.int32),
          pltpu.VMEM((n_chunks, CHUNK), jnp.int32),
          pltpu.VMEM((b_per_w, EMBED), jnp.float32),
          pltpu.VMEM((b_per_w, EMBED), jnp.float32),
          pltpu.SemaphoreType.DMA,
      ],
  )
  def gather(uidx_hbm, midx_hbm, utab_hbm, mtab_hbm, uout_hbm, mout_hbm,
             uidx_v, midx_v, urows_v, mrows_v, sem):
    wid = lax.axis_index("s") * nc + lax.axis_index("c")
    base = wid * b_per_w
    crow = wid * n_chunks
    pltpu.sync_copy(uidx_hbm.at[pl.ds(crow, n_chunks)], uidx_v)
    pltpu.sync_copy(midx_hbm.at[pl.ds(crow, n_chunks)], midx_v)
    copies = []
    for j in range(n_chunks):
      copies.append(pltpu.async_copy(
          utab_hbm.at[uidx_v.at[j]],
          urows_v.at[pl.ds(j * CHUNK, CHUNK)], sem))
      copies.append(pltpu.async_copy(
          mtab_hbm.at[midx_v.at[j]],
          mrows_v.at[pl.ds(j * CHUNK, CHUNK)], sem))
    for c in copies:
      c.wait()
    pltpu.sync_copy(urows_v, uout_hbm.at[pl.ds(base, b_per_w)])
    pltpu.sync_copy(mrows_v, mout_hbm.at[pl.ds(base, b_per_w)])

  del rows_per_chunk
  return gather


def _mlp_body(u_ref, m_ref, w1_ref, b1_ref, w2_ref, b2_ref, w3_ref, b3_ref,
              o_ref):
  h1 = jnp.dot(u_ref[...], w1_ref[0:EMBED, :],
               preferred_element_type=jnp.float32)
  h1 = h1 + jnp.dot(m_ref[...], w1_ref[EMBED:2 * EMBED, :],
                    preferred_element_type=jnp.float32)
  h1 = jnp.maximum(h1 + b1_ref[...], 0.0)
  h2 = jnp.dot(h1, w2_ref[...], preferred_element_type=jnp.float32)
  h2 = jnp.maximum(h2 + b2_ref[...], 0.0)
  o_ref[...] = jnp.sum(h2 * w3_ref[...], axis=1, keepdims=True) + b3_ref[...]


def _mlp_call(u_emb, m_emb, W1, b1, W2, b2, W3, b3):
  bb = 2048
  grid = (BATCH // bb,)
  return pl.pallas_call(
      _mlp_body,
      grid=grid,
      in_specs=[
          pl.BlockSpec((bb, EMBED), lambda i: (i, 0)),
          pl.BlockSpec((bb, EMBED), lambda i: (i, 0)),
          pl.BlockSpec((2 * EMBED, 128), lambda i: (0, 0)),
          pl.BlockSpec((1, 128), lambda i: (0, 0)),
          pl.BlockSpec((128, 64), lambda i: (0, 0)),
          pl.BlockSpec((1, 64), lambda i: (0, 0)),
          pl.BlockSpec((1, 64), lambda i: (0, 0)),
          pl.BlockSpec((1, 1), lambda i: (0, 0)),
      ],
      out_specs=pl.BlockSpec((bb, 1), lambda i: (i, 0)),
      out_shape=jax.ShapeDtypeStruct((BATCH, 1), jnp.float32),
  )(u_emb, m_emb, W1, b1.reshape(1, 128), W2, b2.reshape(1, 64),
    W3.reshape(1, 64), b3.reshape(1, 1))


def kernel(user_input, movie_input, user_table, movie_table,
           W1, b1, W2, b2, W3, b3):
  gather = _make_gather(user_table.shape[0], movie_table.shape[0])
  uidx2 = user_input.reshape(BATCH // CHUNK, CHUNK)
  midx2 = movie_input.reshape(BATCH // CHUNK, CHUNK)
  u_emb, m_emb = gather(uidx2, midx2, user_table, movie_table)
  return _mlp_call(u_emb, m_emb, W1, b1, W2, b2, W3, b3)
